# Initial kernel scaffold; baseline (speedup 1.0000x reference)
import functools

import jax
import jax.numpy as jnp
from jax import lax
from jax.experimental import pallas as pl
from jax.experimental.pallas import tpu as pltpu
from jax.experimental.pallas import tpu_sc as plsc

N_USERS = 25000
N_ITEMS = 25000
N_NODES = N_USERS + N_ITEMS
EMB = 64
HALF = 32
N_LAYERS = 3
E = 800000
NC, NS = 2, 16
EPT = E // NS
SUB = 80
NSUB = 5
CHUNK = SUB * NSUB
NCHUNK = EPT // CHUNK
EROWS = E // SUB
RPT = EPT // SUB
STRIPE = N_NODES // NS
ZCH = 625
CCH = 125


def _body(colr, rowr, valr, t0r, outr, workr,
          accr, zbufr, colbr, rowbr, valbr, rowsr, cb0, cb1, cb2, cb3, sem):
    c = lax.axis_index("c")
    s = lax.axis_index("s")

    zeros = jnp.zeros((16,), jnp.float32)

    def zrow(i, carry):
        zbufr[i, pl.ds(0, 16)] = zeros
        zbufr[i, pl.ds(16, 16)] = zeros
        return carry

    lax.fori_loop(0, ZCH, zrow, 0)

    for layer in range(N_LAYERS):
        if layer == 0:
            src = t0r
            gbase = c * N_NODES
        else:
            src = workr
            gbase = (2 * (layer - 1) + c) * N_NODES

        for k in range(STRIPE // ZCH):
            pltpu.sync_copy(zbufr, accr.at[pl.ds(s * STRIPE + k * ZCH, ZCH), :])
        plsc.subcore_barrier()

        def chunk_body(ci, carry, src=src, gbase=gbase):
            roff = s * RPT + ci * NSUB
            pltpu.sync_copy(colr.at[pl.ds(roff, NSUB), :], colbr)
            pltpu.sync_copy(rowr.at[pl.ds(roff, NSUB), :], rowbr)
            pltpu.sync_copy(valr.at[pl.ds(roff, NSUB), :], valbr)

            def adj(i, carry2):
                r = i // (SUB // 16)
                k16 = (i % (SUB // 16)) * 16
                colbr[r, pl.ds(k16, 16)] = colbr[r, pl.ds(k16, 16)] + gbase
                return carry2

            lax.fori_loop(0, NSUB * (SUB // 16), adj, 0)

            for sub in range(NSUB):
                pltpu.async_copy(src.at[colbr.at[sub]], rowsr, sem).wait()

                def srow(j, carry3, sub=sub):
                    w = valbr[sub, j]
                    rowsr[j, pl.ds(0, 16)] = rowsr[j, pl.ds(0, 16)] * w
                    rowsr[j, pl.ds(16, 16)] = rowsr[j, pl.ds(16, 16)] * w
                    return carry3

                lax.fori_loop(0, SUB, srow, 0)
                pltpu.sync_copy(rowsr, accr.at[rowbr.at[sub]], add=True)
            return carry

        lax.fori_loop(0, NCHUNK, chunk_body, 0)
        plsc.subcore_barrier()

        if layer < N_LAYERS - 1:
            slot = (2 * layer + c) * N_NODES
            pltpu.sync_copy(accr.at[pl.ds(s * STRIPE, STRIPE), :],
                            workr.at[pl.ds(slot + s * STRIPE, STRIPE), :])
            plsc.subcore_barrier()

    for k in range(STRIPE // CCH):
        r0 = s * STRIPE + k * CCH
        pltpu.sync_copy(t0r.at[pl.ds(c * N_NODES + r0, CCH), :], cb0)
        pltpu.sync_copy(workr.at[pl.ds(c * N_NODES + r0, CCH), :], cb1)
        pltpu.sync_copy(workr.at[pl.ds((2 + c) * N_NODES + r0, CCH), :], cb2)
        pltpu.sync_copy(accr.at[pl.ds(r0, CCH), :], cb3)

        def crow(j, carry):
            for k16 in (0, 16):
                v = (cb0[j, pl.ds(k16, 16)] + cb1[j, pl.ds(k16, 16)]
                     + cb2[j, pl.ds(k16, 16)] + cb3[j, pl.ds(k16, 16)])
                cb0[j, pl.ds(k16, 16)] = v * 0.25
            return carry

        lax.fori_loop(0, CCH, crow, 0)
        pltpu.sync_copy(cb0, outr.at[c, pl.ds(r0, CCH), :])


_lightgcn = functools.partial(
    pl.kernel,
    out_type=(
        jax.ShapeDtypeStruct((NC, N_NODES, HALF), jnp.float32),
        jax.ShapeDtypeStruct((4 * N_NODES, HALF), jnp.float32),
    ),
    mesh=plsc.VectorSubcoreMesh(core_axis_name="c", subcore_axis_name="s",
                                num_cores=NC, num_subcores=NS),
    scratch_types=[
        pltpu.VMEM_SHARED((N_NODES, HALF), jnp.float32),
        pltpu.VMEM((ZCH, HALF), jnp.float32),
        pltpu.VMEM((NSUB, SUB), jnp.int32),
        pltpu.VMEM((NSUB, SUB), jnp.int32),
        pltpu.VMEM((NSUB, SUB), jnp.float32),
        pltpu.VMEM((SUB, HALF), jnp.float32),
        pltpu.VMEM((CCH, HALF), jnp.float32),
        pltpu.VMEM((CCH, HALF), jnp.float32),
        pltpu.VMEM((CCH, HALF), jnp.float32),
        pltpu.VMEM((CCH, HALF), jnp.float32),
        pltpu.SemaphoreType.DMA,
    ],
)(_body)


def kernel(adj_indices, adj_values, user_table, item_table):
    col = adj_indices[1].astype(jnp.int32).reshape(EROWS, SUB)
    row = adj_indices[0].astype(jnp.int32).reshape(EROWS, SUB)
    vals = adj_values.reshape(EROWS, SUB)
    all_emb = jnp.concatenate([user_table, item_table], axis=0)
    t0 = jnp.stack([all_emb[:, :HALF], all_emb[:, HALF:]], axis=0)
    t0 = t0.reshape(NC * N_NODES, HALF)
    out, _ = _lightgcn(col, row, vals, t0)
    light = jnp.concatenate([out[0], out[1]], axis=1)
    return (light[:N_USERS], light[N_USERS:])


# speedup vs baseline: 3.7379x; 3.7379x over previous
import functools

import jax
import jax.numpy as jnp
from jax import lax
from jax.experimental import pallas as pl
from jax.experimental.pallas import tpu as pltpu
from jax.experimental.pallas import tpu_sc as plsc

N_USERS = 25000
N_ITEMS = 25000
N_NODES = N_USERS + N_ITEMS
HALF = 32
N_LAYERS = 3
E = 800000
NC, NS = 2, 16
SUB = 80
NSUB = 8
E_PAD = 819200
EROWS = E_PAD // SUB
RPT = EROWS // NS
NCHUNK = RPT // NSUB
N_PAD = 50176
STRIPE = N_PAD // NS
ZCH = 112
CCH = 112


def _body(colr, rowr, valr, t0r, outr, workr,
          accr, zbufr, colbr, rowbr, valbr, rowsr, cb0, cb1, cb2, cb3, sem):
    c = lax.axis_index("c")
    s = lax.axis_index("s")

    zeros = jnp.zeros((16,), jnp.float32)

    def zrow(i, carry):
        zbufr[i, pl.ds(0, 16)] = zeros
        zbufr[i, pl.ds(16, 16)] = zeros
        return carry

    lax.fori_loop(0, ZCH, zrow, 0)

    for layer in range(N_LAYERS):
        if layer == 0:
            src = t0r
            gbase = c * N_PAD
        else:
            src = workr
            gbase = (2 * (layer - 1) + c) * N_PAD

        for k in range(STRIPE // ZCH):
            pltpu.sync_copy(zbufr, accr.at[pl.ds(s * STRIPE + k * ZCH, ZCH), :])
        plsc.subcore_barrier()

        def chunk_body(ci, carry, src=src, gbase=gbase):
            roff = s * RPT + ci * NSUB
            pltpu.sync_copy(colr.at[pl.ds(roff, NSUB), :], colbr)
            pltpu.sync_copy(rowr.at[pl.ds(roff, NSUB), :], rowbr)
            pltpu.sync_copy(valr.at[pl.ds(roff, NSUB), :], valbr)

            for r in range(NSUB):
                for k in range(SUB // 16):
                    colbr[r, pl.ds(k * 16, 16)] = (
                        colbr[r, pl.ds(k * 16, 16)] + gbase)

            for sub in range(NSUB):
                pltpu.async_copy(src.at[colbr.at[sub]], rowsr, sem).wait()

                def sgrp(g, carry3, sub=sub):
                    wv = valbr[sub, pl.ds(g * 16, 16)]
                    for l in range(16):
                        w = wv[l]
                        j = g * 16 + l
                        rowsr[j, pl.ds(0, 16)] = rowsr[j, pl.ds(0, 16)] * w
                        rowsr[j, pl.ds(16, 16)] = rowsr[j, pl.ds(16, 16)] * w
                    return carry3

                lax.fori_loop(0, SUB // 16, sgrp, 0)
                pltpu.sync_copy(rowsr, accr.at[rowbr.at[sub]], add=True)
            return carry

        lax.fori_loop(0, NCHUNK, chunk_body, 0)
        plsc.subcore_barrier()

        if layer < N_LAYERS - 1:
            slot = (2 * layer + c) * N_PAD
            pltpu.sync_copy(accr.at[pl.ds(s * STRIPE, STRIPE), :],
                            workr.at[pl.ds(slot + s * STRIPE, STRIPE), :])
            plsc.subcore_barrier()

    for k in range(STRIPE // CCH):
        r0 = s * STRIPE + k * CCH
        pltpu.sync_copy(t0r.at[pl.ds(c * N_PAD + r0, CCH), :], cb0)
        pltpu.sync_copy(workr.at[pl.ds(c * N_PAD + r0, CCH), :], cb1)
        pltpu.sync_copy(workr.at[pl.ds((2 + c) * N_PAD + r0, CCH), :], cb2)
        pltpu.sync_copy(accr.at[pl.ds(r0, CCH), :], cb3)

        def crow(j, carry):
            for k16 in (0, 16):
                v = (cb0[j, pl.ds(k16, 16)] + cb1[j, pl.ds(k16, 16)]
                     + cb2[j, pl.ds(k16, 16)] + cb3[j, pl.ds(k16, 16)])
                cb0[j, pl.ds(k16, 16)] = v * 0.25
            return carry

        lax.fori_loop(0, CCH, crow, 0)
        pltpu.sync_copy(cb0, outr.at[c, pl.ds(r0, CCH), :])


_lightgcn = functools.partial(
    pl.kernel,
    out_type=(
        jax.ShapeDtypeStruct((NC, N_PAD, HALF), jnp.float32),
        jax.ShapeDtypeStruct((4 * N_PAD, HALF), jnp.float32),
    ),
    mesh=plsc.VectorSubcoreMesh(core_axis_name="c", subcore_axis_name="s",
                                num_cores=NC, num_subcores=NS),
    scratch_types=[
        pltpu.VMEM_SHARED((N_PAD, HALF), jnp.float32),
        pltpu.VMEM((ZCH, HALF), jnp.float32),
        pltpu.VMEM((NSUB, SUB), jnp.int32),
        pltpu.VMEM((NSUB, SUB), jnp.int32),
        pltpu.VMEM((NSUB, SUB), jnp.float32),
        pltpu.VMEM((SUB, HALF), jnp.float32),
        pltpu.VMEM((CCH, HALF), jnp.float32),
        pltpu.VMEM((CCH, HALF), jnp.float32),
        pltpu.VMEM((CCH, HALF), jnp.float32),
        pltpu.VMEM((CCH, HALF), jnp.float32),
        pltpu.SemaphoreType.DMA,
    ],
    compiler_params=pltpu.CompilerParams(use_tc_tiling_on_sc=False),
)(_body)


def kernel(adj_indices, adj_values, user_table, item_table):
    pad_e = E_PAD - E
    col = jnp.concatenate(
        [adj_indices[1].astype(jnp.int32), jnp.zeros((pad_e,), jnp.int32)])
    row = jnp.concatenate(
        [adj_indices[0].astype(jnp.int32), jnp.zeros((pad_e,), jnp.int32)])
    vals = jnp.concatenate([adj_values, jnp.zeros((pad_e,), jnp.float32)])
    col = col.reshape(EROWS, SUB)
    row = row.reshape(EROWS, SUB)
    vals = vals.reshape(EROWS, SUB)
    all_emb = jnp.concatenate([user_table, item_table], axis=0)
    all_emb = jnp.pad(all_emb, ((0, N_PAD - N_NODES), (0, 0)))
    t0 = jnp.stack([all_emb[:, :HALF], all_emb[:, HALF:]], axis=0)
    t0 = t0.reshape(NC * N_PAD, HALF)
    out, _ = _lightgcn(col, row, vals, t0)
    light = jnp.concatenate([out[0, :N_NODES], out[1, :N_NODES]], axis=1)
    return (light[:N_USERS], light[N_USERS:])


# speedup vs baseline: 7.2238x; 1.9326x over previous
import functools

import jax
import jax.numpy as jnp
from jax import lax
from jax.experimental import pallas as pl
from jax.experimental.pallas import tpu as pltpu
from jax.experimental.pallas import tpu_sc as plsc

N_USERS = 25000
N_ITEMS = 25000
N_NODES = N_USERS + N_ITEMS
HALF = 32
N_LAYERS = 3
E = 800000
NC, NS = 2, 16
SUB = 128
NSUB = 8
E_PAD = 819200
EROWS = E_PAD // SUB
RPT = EROWS // NS
NCHUNK = RPT // NSUB
N_PAD = 50176
STRIPE = N_PAD // NS
CCH = 112


def _body(colr, rowr, valr, t0r, outr, workr,
          accr, colbr, rowbr, valbr, rb0, rb1, rb2, cb0, cb1,
          gs0, gs1, gs2, ss0, ss1, ss2, ec0, ec1, er0, er1, ev0, ev1,
          csem0, csem1, csem2, csem3, stsem):
    c = lax.axis_index("c")
    s = lax.axis_index("s")
    rbufs = (rb0, rb1, rb2)
    gsems = (gs0, gs1, gs2)
    ssems = (ss0, ss1, ss2)
    esems = ((ec0, er0, ev0), (ec1, er1, ev1))

    zeros = jnp.zeros((16,), jnp.float32)

    def zrow(i, carry):
        cb0[i, pl.ds(0, 16)] = zeros
        cb0[i, pl.ds(16, 16)] = zeros
        return carry

    lax.fori_loop(0, CCH, zrow, 0)

    def issue_edge_loads(roff, p):
        pltpu.async_copy(colr.at[pl.ds(roff, NSUB), :], colbr.at[p], esems[p][0])
        pltpu.async_copy(rowr.at[pl.ds(roff, NSUB), :], rowbr.at[p], esems[p][1])
        pltpu.async_copy(valr.at[pl.ds(roff, NSUB), :], valbr.at[p], esems[p][2])

    def wait_edge_loads(p):
        pltpu.make_async_copy(colr.at[pl.ds(0, NSUB), :], colbr.at[p],
                              esems[p][0]).wait()
        pltpu.make_async_copy(rowr.at[pl.ds(0, NSUB), :], rowbr.at[p],
                              esems[p][1]).wait()
        pltpu.make_async_copy(valr.at[pl.ds(0, NSUB), :], valbr.at[p],
                              esems[p][2]).wait()

    def scale(rb, p, i):
        def sgrp(g, carry):
            wv = valbr[p, i, pl.ds(g * 16, 16)]
            for l in range(16):
                w = wv[l]
                j = g * 16 + l
                rb[j, pl.ds(0, 16)] = rb[j, pl.ds(0, 16)] * w
                rb[j, pl.ds(16, 16)] = rb[j, pl.ds(16, 16)] * w
            return carry

        lax.fori_loop(0, SUB // 16, sgrp, 0)

    def do_chunk(src, gbase, p):
        for r in range(NSUB):
            for k in range(SUB // 16):
                colbr[p, r, pl.ds(k * 16, 16)] = (
                    colbr[p, r, pl.ds(k * 16, 16)] + gbase)
        gd = [None] * NSUB
        sd = [None] * NSUB
        gd[0] = pltpu.async_copy(src.at[colbr.at[p, 0]], rbufs[0], gsems[0])
        gd[1] = pltpu.async_copy(src.at[colbr.at[p, 1]], rbufs[1], gsems[1])
        for i in range(NSUB):
            sl = i % 3
            gd[i].wait()
            scale(rbufs[sl], p, i)
            if i + 2 < NSUB:
                if i >= 1:
                    sd[i - 1].wait()
                nsl = (i + 2) % 3
                gd[i + 2] = pltpu.async_copy(src.at[colbr.at[p, i + 2]],
                                             rbufs[nsl], gsems[nsl])
            sd[i] = pltpu.async_copy(rbufs[sl], accr.at[rowbr.at[p, i]],
                                     ssems[sl], add=True)
        sd[NSUB - 3].wait()
        sd[NSUB - 2].wait()
        sd[NSUB - 1].wait()

    for layer in range(N_LAYERS):
        if layer == 0:
            src = t0r
            gbase = c * N_PAD
        else:
            src = workr
            gbase = (2 * (layer - 1) + c) * N_PAD

        issue_edge_loads(s * RPT, 0)
        for k in range(STRIPE // CCH):
            pltpu.sync_copy(cb0, accr.at[pl.ds(s * STRIPE + k * CCH, CCH), :])
        plsc.subcore_barrier()

        def pair_body(k, carry, src=src, gbase=gbase):
            wait_edge_loads(0)
            issue_edge_loads(s * RPT + (2 * k + 1) * NSUB, 1)
            do_chunk(src, gbase, 0)
            wait_edge_loads(1)

            @pl.when(k < NCHUNK // 2 - 1)
            def _():
                issue_edge_loads(s * RPT + (2 * k + 2) * NSUB, 0)

            do_chunk(src, gbase, 1)
            return carry

        lax.fori_loop(0, NCHUNK // 2, pair_body, 0)
        plsc.subcore_barrier()

        if layer < N_LAYERS - 1:
            slot = (2 * layer + c) * N_PAD
            pltpu.sync_copy(accr.at[pl.ds(s * STRIPE, STRIPE), :],
                            workr.at[pl.ds(slot + s * STRIPE, STRIPE), :])
            plsc.subcore_barrier()

    csems = (csem0, csem1, csem2, csem3)
    for k in range(STRIPE // CCH):
        r0 = s * STRIPE + k * CCH
        l0 = pltpu.async_copy(t0r.at[pl.ds(c * N_PAD + r0, CCH), :],
                              rb0.at[pl.ds(0, CCH), :], csem0)
        l1 = pltpu.async_copy(workr.at[pl.ds(c * N_PAD + r0, CCH), :],
                              rb1.at[pl.ds(0, CCH), :], csem1)
        l2 = pltpu.async_copy(workr.at[pl.ds((2 + c) * N_PAD + r0, CCH), :],
                              rb2.at[pl.ds(0, CCH), :], csem2)
        l3 = pltpu.async_copy(accr.at[pl.ds(r0, CCH), :], cb1, csem3)
        l0.wait()
        l1.wait()
        l2.wait()
        l3.wait()
        if k > 0:
            pltpu.make_async_copy(cb0, outr.at[c, pl.ds(0, CCH), :],
                                  stsem).wait()

        def crow(j, carry):
            for k16 in (0, 16):
                v = (rb0[j, pl.ds(k16, 16)] + rb1[j, pl.ds(k16, 16)]
                     + rb2[j, pl.ds(k16, 16)] + cb1[j, pl.ds(k16, 16)])
                cb0[j, pl.ds(k16, 16)] = v * 0.25
            return carry

        lax.fori_loop(0, CCH, crow, 0)
        pltpu.async_copy(cb0, outr.at[c, pl.ds(r0, CCH), :], stsem)
    pltpu.make_async_copy(cb0, outr.at[c, pl.ds(0, CCH), :], stsem).wait()


_lightgcn = functools.partial(
    pl.kernel,
    out_type=(
        jax.ShapeDtypeStruct((NC, N_PAD, HALF), jnp.float32),
        jax.ShapeDtypeStruct((4 * N_PAD, HALF), jnp.float32),
    ),
    mesh=plsc.VectorSubcoreMesh(core_axis_name="c", subcore_axis_name="s",
                                num_cores=NC, num_subcores=NS),
    scratch_types=[
        pltpu.VMEM_SHARED((N_PAD, HALF), jnp.float32),
        pltpu.VMEM((2, NSUB, SUB), jnp.int32),
        pltpu.VMEM((2, NSUB, SUB), jnp.int32),
        pltpu.VMEM((2, NSUB, SUB), jnp.float32),
        pltpu.VMEM((SUB, HALF), jnp.float32),
        pltpu.VMEM((SUB, HALF), jnp.float32),
        pltpu.VMEM((SUB, HALF), jnp.float32),
        pltpu.VMEM((CCH, HALF), jnp.float32),
        pltpu.VMEM((CCH, HALF), jnp.float32),
        pltpu.SemaphoreType.DMA,
        pltpu.SemaphoreType.DMA,
        pltpu.SemaphoreType.DMA,
        pltpu.SemaphoreType.DMA,
        pltpu.SemaphoreType.DMA,
        pltpu.SemaphoreType.DMA,
        pltpu.SemaphoreType.DMA,
        pltpu.SemaphoreType.DMA,
        pltpu.SemaphoreType.DMA,
        pltpu.SemaphoreType.DMA,
        pltpu.SemaphoreType.DMA,
        pltpu.SemaphoreType.DMA,
        pltpu.SemaphoreType.DMA,
        pltpu.SemaphoreType.DMA,
        pltpu.SemaphoreType.DMA,
        pltpu.SemaphoreType.DMA,
        pltpu.SemaphoreType.DMA,
    ],
    compiler_params=pltpu.CompilerParams(use_tc_tiling_on_sc=False),
)(_body)


def kernel(adj_indices, adj_values, user_table, item_table):
    pad_e = E_PAD - E
    col = jnp.concatenate(
        [adj_indices[1].astype(jnp.int32), jnp.zeros((pad_e,), jnp.int32)])
    row = jnp.concatenate(
        [adj_indices[0].astype(jnp.int32), jnp.zeros((pad_e,), jnp.int32)])
    vals = jnp.concatenate([adj_values, jnp.zeros((pad_e,), jnp.float32)])
    col = col.reshape(EROWS, SUB)
    row = row.reshape(EROWS, SUB)
    vals = vals.reshape(EROWS, SUB)
    all_emb = jnp.concatenate([user_table, item_table], axis=0)
    all_emb = jnp.pad(all_emb, ((0, N_PAD - N_NODES), (0, 0)))
    t0 = jnp.stack([all_emb[:, :HALF], all_emb[:, HALF:]], axis=0)
    t0 = t0.reshape(NC * N_PAD, HALF)
    out, _ = _lightgcn(col, row, vals, t0)
    light = jnp.concatenate([out[0, :N_NODES], out[1, :N_NODES]], axis=1)
    return (light[:N_USERS], light[N_USERS:])
